# baseline (speedup 1.0000x reference)
import functools

import jax
import jax.numpy as jnp
from jax import lax
from jax.experimental import pallas as pl
from jax.experimental.pallas import tpu as pltpu
from jax.experimental.pallas import tpu_sc as plsc


def _make_transpose_kernel(V, P, NC, NS):
    NW = NC * NS
    TB = 512
    NFULL = (V // 128) * 128
    NUNIT = NFULL // TB
    NU = (NUNIT // NW) // 2 * 2
    NEXTRA = NUNIT - NU * NW
    TAIL = V - NFULL
    OW = TB * P
    assert NU % 2 == 0

    mesh = plsc.VectorSubcoreMesh(core_axis_name="c", subcore_axis_name="s")

    @functools.partial(
        pl.kernel,
        out_type=jax.ShapeDtypeStruct((V * P,), jnp.float32),
        mesh=mesh,
        scratch_types=[
            pltpu.VMEM((P, TB), jnp.float32),
            pltpu.VMEM((P, TB), jnp.float32),
            pltpu.VMEM((OW,), jnp.float32),
            pltpu.VMEM((OW,), jnp.float32),
            pltpu.VMEM((TAIL * P,), jnp.float32),
            pltpu.SemaphoreType.DMA,
            pltpu.SemaphoreType.DMA,
            pltpu.SemaphoreType.DMA,
            pltpu.SemaphoreType.DMA,
        ],
        compiler_params=pltpu.CompilerParams(
            needs_layout_passes=False, use_tc_tiling_on_sc=True),
    )
    def tr_kernel(nt_hbm, tail_hbm, out_hbm, blk0, blk1, ob0, ob1, tbuf,
                  sin0, sin1, sout0, sout1):
        wid = lax.axis_index("s") * NC + lax.axis_index("c")
        iota16 = lax.iota(jnp.int32, 16)
        iotaP = iota16 * P
        base = wid * NU

        colps = [lax.bitwise_and(iota16 + g, P - 1) for g in range(P)]

        def transpose(blk, ob):
            def lg_body(lg, _):
                lidx = lg * 16 + iota16
                lP = lidx * P
                for g in range(P):
                    v = plsc.load_gather(blk, [colps[g], lidx])
                    plsc.store_scatter(ob, [lP + colps[g]], v)
                return 0

            lax.fori_loop(0, TB // 16, lg_body, 0)

        def in_desc(blk, i, sin):
            return pltpu.make_async_copy(
                nt_hbm.at[:, pl.ds((base + i) * TB, TB)], blk, sin)

        def out_desc(ob, i, sout):
            return pltpu.make_async_copy(
                ob, out_hbm.at[pl.ds((base + i) * TB * P, OW)], sout)

        in_desc(blk0, 0, sin0).start()
        in_desc(blk1, 1, sin1).start()

        def pair_body(g, _):
            for blk, ob, sin, sout in ((blk0, ob0, sin0, sout0),
                                       (blk1, ob1, sin1, sout1)):
                i = 2 * g + (0 if blk is blk0 else 1)
                in_desc(blk, i, sin).wait()

                @pl.when(g > 0)
                def _():
                    out_desc(ob, i, sout).wait()

                transpose(blk, ob)
                out_desc(ob, i, sout).start()

                @pl.when(g < NU // 2 - 1)
                def _():
                    in_desc(blk, i + 2, sin).start()

            return 0

        lax.fori_loop(0, NU // 2, pair_body, 0)
        out_desc(ob0, 0, sout0).wait()
        out_desc(ob1, 1, sout1).wait()

        for e in range((NEXTRA + NW - 1) // NW):
            u = NU * NW + e * NW + wid

            @pl.when(u < NUNIT)
            def _():
                c0 = u * TB
                pltpu.sync_copy(nt_hbm.at[:, pl.ds(c0, TB)], blk0)
                transpose(blk0, ob0)
                pltpu.sync_copy(ob0, out_hbm.at[pl.ds(c0 * P, OW)])

        @pl.when(wid == NW - 1)
        def _():
            pltpu.sync_copy(tail_hbm, tbuf)
            pltpu.sync_copy(tbuf, out_hbm.at[pl.ds(NFULL * P, TAIL * P)])

    return tr_kernel


def _make_dot_kernel(B, K, V, P, NC, NS):
    NW = NC * NS
    BPW = B // NW
    C = 32
    NCHUNK = BPW // C
    J = C * K
    GSUB = 112
    NSUB = J // GSUB
    NG = C // 16
    NFLAT = J // 16
    MAGIC = (1 << 16) // K + 1

    mesh = plsc.VectorSubcoreMesh(core_axis_name="c", subcore_axis_name="s")

    @functools.partial(
        pl.kernel,
        out_type=jax.ShapeDtypeStruct((B, 128), jnp.float32),
        mesh=mesh,
        scratch_types=[
            pltpu.VMEM((C, 128), jnp.int32),
            pltpu.VMEM((J,), jnp.int32),
            pltpu.VMEM((J,), jnp.int32),
            pltpu.VMEM((C, 128), jnp.float32),
            pltpu.VMEM((C, 128), jnp.float32),
            pltpu.VMEM((J, P), jnp.float32),
            pltpu.VMEM((J, P), jnp.float32),
            pltpu.VMEM((C, 128), jnp.float32),
            pltpu.SemaphoreType.DMA,
            pltpu.SemaphoreType.DMA,
        ],
        compiler_params=pltpu.CompilerParams(
            needs_layout_passes=False, use_tc_tiling_on_sc=False),
    )
    def dot_kernel(s_hbm, e_hbm, nembt_hbm, out_hbm,
                   sblk, sflat0, sflat1, erows0, erows1,
                   nrows0, nrows1, outv, sem0, sem1):
        wid = lax.axis_index("s") * NC + lax.axis_index("c")
        iota16 = lax.iota(jnp.int32, 16)
        bufs = ((sblk, sflat0, erows0, nrows0, outv, sem0),
                (sblk, sflat1, erows1, nrows1, outv, sem1))

        def gather_descs(ci, buf):
            sblk, sflat, erows, nrows, outv, sem = buf
            nbase = wid * BPW + ci * C
            ds = [pltpu.make_async_copy(e_hbm.at[pl.ds(nbase, C)], erows, sem)]
            for s in range(NSUB):
                ds.append(pltpu.make_async_copy(
                    nembt_hbm.at[sflat.at[pl.ds(s * GSUB, GSUB)]],
                    nrows.at[pl.ds(s * GSUB, GSUB)], sem))
            return ds

        def stage(ci, buf):
            sblk, sflat, erows, nrows, outv, sem = buf
            nbase = wid * BPW + ci * C
            pltpu.sync_copy(s_hbm.at[pl.ds(nbase, C)], sblk)

            def flat_body(g, _):
                j16 = g * 16 + iota16
                q = lax.shift_right_logical(j16 * MAGIC, 16)
                r = j16 - q * K
                sflat[pl.ds(g * 16, 16)] = plsc.load_gather(sblk, [q, r])
                return 0

            lax.fori_loop(0, NFLAT, flat_body, 0)
            for d in gather_descs(ci, buf):
                d.start()

        def compute(ci, buf):
            sblk, sflat, erows, nrows, outv, sem = buf
            nbase = wid * BPW + ci * C

            def group_body(g, _):
                nloc = g * 16 + iota16
                evd = [plsc.load_gather(
                           erows, [nloc, lax.bitwise_and(iota16 + c, P - 1)])
                       for c in range(P)]
                for k in range(K):
                    row16 = nloc * K + k
                    acc = evd[0] * plsc.load_gather(
                        nrows, [row16, lax.bitwise_and(iota16, P - 1)])
                    for c in range(1, P):
                        acc = acc + evd[c] * plsc.load_gather(
                            nrows, [row16, lax.bitwise_and(iota16 + c, P - 1)])
                    plsc.store_scatter(outv, [nloc, jnp.full((16,), k, jnp.int32)], acc)
                return 0

            lax.fori_loop(0, NG, group_body, 0)
            pltpu.sync_copy(outv, out_hbm.at[pl.ds(nbase, C)])

        stage(0, bufs[0])
        stage(1, bufs[1])

        def pair_body(g, _):
            for b in (0, 1):
                ci = 2 * g + b
                for d in gather_descs(ci, bufs[b]):
                    d.wait()
                compute(ci, bufs[b])

                @pl.when(g < NCHUNK // 2 - 1)
                def _():
                    stage(ci + 2, bufs[b])

            return 0

        lax.fori_loop(0, NCHUNK // 2, pair_body, 0)

    return dot_kernel


def kernel(x, sampled, Emb, NEmb):
    B = x.shape[0]
    K = sampled.shape[1]
    V, P = Emb.shape
    try:
        info = plsc.get_sparse_core_info()
        NC, NS = info.num_cores, info.num_subcores
    except Exception:
        NC, NS = 2, 16
    NFULL = (V // 128) * 128

    tr = _make_transpose_kernel(V, P, NC, NS)
    dot = _make_dot_kernel(B, K, V, P, NC, NS)

    nt = NEmb.T
    tailf = NEmb[NFULL:, :].reshape(-1)
    nembt = tr(nt, tailf).reshape(V, P)

    e = jnp.take(Emb, x, axis=0)
    e128 = jnp.pad(e, ((0, 0), (0, 128 - P)))

    spad = jnp.pad(sampled, ((0, 0), (0, 128 - K)))
    out = dot(spad, e128, nembt)
    return out[:, :K]


# speedup vs baseline: 1.0652x; 1.0652x over previous
import functools

import jax
import jax.numpy as jnp
from jax import lax
from jax.experimental import pallas as pl
from jax.experimental.pallas import tpu as pltpu
from jax.experimental.pallas import tpu_sc as plsc


def _make_transpose_kernel(V, P, NC, NS):
    NW = NC * NS
    TB = 256
    NFULL = (V // 128) * 128
    NUNIT = NFULL // TB
    NU = NUNIT // NW
    NEXTRA = NUNIT - NU * NW
    TAIL = V - NFULL
    OW = TB * P
    assert NU % 2 == 0

    mesh = plsc.VectorSubcoreMesh(core_axis_name="c", subcore_axis_name="s")

    @functools.partial(
        pl.kernel,
        out_type=jax.ShapeDtypeStruct((V * P,), jnp.float32),
        mesh=mesh,
        scratch_types=[
            pltpu.VMEM((P, TB), jnp.float32),
            pltpu.VMEM((P, TB), jnp.float32),
            pltpu.VMEM((OW,), jnp.float32),
            pltpu.VMEM((OW,), jnp.float32),
            pltpu.VMEM((TAIL * P,), jnp.float32),
            pltpu.SemaphoreType.DMA,
            pltpu.SemaphoreType.DMA,
        ],
        compiler_params=pltpu.CompilerParams(
            needs_layout_passes=False, use_tc_tiling_on_sc=True),
    )
    def tr_kernel(nt_hbm, tail_hbm, out_hbm, blk0, blk1, ob0, ob1, tbuf,
                  sin, sout):
        wid = lax.axis_index("s") * NC + lax.axis_index("c")
        iota16 = lax.iota(jnp.int32, 16)
        iotaP = iota16 * P
        base = wid * NU

        colps = [lax.bitwise_and(iota16 + g, P - 1) for g in range(P)]

        def transpose(blk, ob):
            def lg_body(lg, _):
                lidx = lg * 16 + iota16
                lP = lidx * P
                for g in range(P):
                    v = plsc.load_gather(blk, [colps[g], lidx])
                    plsc.store_scatter(ob, [lP + colps[g]], v)
                return 0

            lax.fori_loop(0, TB // 16, lg_body, 0)

        def in_desc(blk, i):
            return pltpu.make_async_copy(
                nt_hbm.at[:, pl.ds((base + i) * TB, TB)], blk, sin)

        def out_desc(ob, i):
            return pltpu.make_async_copy(
                ob, out_hbm.at[pl.ds((base + i) * TB * P, OW)], sout)

        in_desc(blk0, 0).start()
        in_desc(blk1, 1).start()

        def pair_body(g, _):
            for b, blk, ob in ((0, blk0, ob0), (1, blk1, ob1)):
                i = 2 * g + b
                in_desc(blk, i).wait()

                @pl.when(g > 0)
                def _():
                    out_desc(ob, i).wait()

                transpose(blk, ob)
                out_desc(ob, i).start()

                @pl.when(g < NU // 2 - 1)
                def _():
                    in_desc(blk, i + 2).start()

            return 0

        lax.fori_loop(0, NU // 2, pair_body, 0)
        out_desc(ob0, 0).wait()
        out_desc(ob1, 1).wait()

        @pl.when(wid < NEXTRA)
        def _():
            c0 = (NU * NW + wid) * TB
            pltpu.sync_copy(nt_hbm.at[:, pl.ds(c0, TB)], blk0)
            transpose(blk0, ob0)
            pltpu.sync_copy(ob0, out_hbm.at[pl.ds(c0 * P, OW)])

        @pl.when(wid == NW - 1)
        def _():
            pltpu.sync_copy(tail_hbm, tbuf)
            pltpu.sync_copy(tbuf, out_hbm.at[pl.ds(NFULL * P, TAIL * P)])

    return tr_kernel


def _make_dot_kernel(B, K, V, P, NC, NS):
    NW = NC * NS
    BPW = B // NW
    C = 64
    NCHUNK = BPW // C
    J = C * K
    GSUB = 112
    NSUB = J // GSUB
    NG = C // 16
    NFLAT = J // 16
    MAGIC = (1 << 16) // K + 1

    mesh = plsc.VectorSubcoreMesh(core_axis_name="c", subcore_axis_name="s")

    @functools.partial(
        pl.kernel,
        out_type=jax.ShapeDtypeStruct((B, 128), jnp.float32),
        mesh=mesh,
        scratch_types=[
            pltpu.VMEM((C, 128), jnp.int32),
            pltpu.VMEM((J,), jnp.int32),
            pltpu.VMEM((C, 128), jnp.float32),
            pltpu.VMEM((J, P), jnp.float32),
            pltpu.VMEM((C, 128), jnp.float32),
            pltpu.SemaphoreType.DMA,
        ],
        compiler_params=pltpu.CompilerParams(
            needs_layout_passes=False, use_tc_tiling_on_sc=False),
    )
    def dot_kernel(s_hbm, e_hbm, nembt_hbm, out_hbm,
                   sblk, sflat, erows, nrows, outv, sem):
        wid = lax.axis_index("s") * NC + lax.axis_index("c")
        iota16 = lax.iota(jnp.int32, 16)

        def chunk_body(ci, _):
            nbase = wid * BPW + ci * C

            pltpu.sync_copy(s_hbm.at[pl.ds(nbase, C)], sblk)
            edesc = pltpu.async_copy(e_hbm.at[pl.ds(nbase, C)], erows, sem)

            def flat_body(g, _):
                j16 = g * 16 + iota16
                q = lax.shift_right_logical(j16 * MAGIC, 16)
                r = j16 - q * K
                sflat[pl.ds(g * 16, 16)] = plsc.load_gather(sblk, [q, r])
                return 0

            lax.fori_loop(0, NFLAT, flat_body, 0)

            descs = [edesc]
            for s in range(NSUB):
                descs.append(
                    pltpu.async_copy(nembt_hbm.at[sflat.at[pl.ds(s * GSUB, GSUB)]],
                                     nrows.at[pl.ds(s * GSUB, GSUB)], sem))
            for d in descs:
                d.wait()

            def group_body(g, _):
                nloc = g * 16 + iota16
                evd = [plsc.load_gather(
                           erows, [nloc, lax.bitwise_and(iota16 + c, P - 1)])
                       for c in range(P)]
                for k in range(K):
                    row16 = nloc * K + k
                    acc = evd[0] * plsc.load_gather(
                        nrows, [row16, lax.bitwise_and(iota16, P - 1)])
                    for c in range(1, P):
                        acc = acc + evd[c] * plsc.load_gather(
                            nrows, [row16, lax.bitwise_and(iota16 + c, P - 1)])
                    plsc.store_scatter(outv, [nloc, jnp.full((16,), k, jnp.int32)], acc)
                return 0

            lax.fori_loop(0, NG, group_body, 0)
            pltpu.sync_copy(outv, out_hbm.at[pl.ds(nbase, C)])
            return 0

        lax.fori_loop(0, NCHUNK, chunk_body, 0)

    return dot_kernel


def kernel(x, sampled, Emb, NEmb):
    B = x.shape[0]
    K = sampled.shape[1]
    V, P = Emb.shape
    try:
        info = plsc.get_sparse_core_info()
        NC, NS = info.num_cores, info.num_subcores
    except Exception:
        NC, NS = 2, 16
    NFULL = (V // 128) * 128

    tr = _make_transpose_kernel(V, P, NC, NS)
    dot = _make_dot_kernel(B, K, V, P, NC, NS)

    nt = NEmb.T
    tailf = NEmb[NFULL:, :].reshape(-1)
    nembt = tr(nt, tailf).reshape(V, P)

    e = jnp.take(Emb, x, axis=0)
    e128 = jnp.pad(e, ((0, 0), (0, 128 - P)))

    spad = jnp.pad(sampled, ((0, 0), (0, 128 - K)))
    out = dot(spad, e128, nembt)
    return out[:, :K]
